# baseline (speedup 1.0000x reference)
import functools

import jax
import jax.numpy as jnp
from jax import lax
from jax.experimental import pallas as pl
from jax.experimental.pallas import tpu as pltpu
from jax.experimental.pallas import tpu_sc as plsc

N = 10000
NP = 10112
E = 320000
EB = 128
NW = 32
WB = 80
PH = 40
E_PAD = NW * WB * EB
RPT = NP // 16
DEG_PAD = 10240
DEG_RPT = DEG_PAD // 16

_MESH = plsc.VectorSubcoreMesh(
    core_axis_name="c", subcore_axis_name="s", num_cores=2, num_subcores=16
)


def _make_prop(feat):
    @functools.partial(
        pl.kernel,
        mesh=_MESH,
        out_type=(jax.ShapeDtypeStruct((NP, feat), jnp.float32),) * 2,
        scratch_types=[
            pltpu.VMEM((PH, EB), jnp.int32),
            pltpu.VMEM((PH, EB), jnp.int32),
            pltpu.VMEM((EB, feat), jnp.float32),
            pltpu.VMEM((EB, feat), jnp.float32),
            pltpu.VMEM_SHARED((NP, feat), jnp.float32),
            pltpu.SemaphoreType.DMA,
            pltpu.SemaphoreType.DMA,
            pltpu.SemaphoreType.DMA,
        ],
    )
    def prop(src_hbm, dst_hbm, feat_hbm, zeros_hbm, o0, o1,
             srcs, dsts, r0, r1, acc, isem, gsem0, gsem1):
        c = lax.axis_index("c")
        s = lax.axis_index("s")
        w = c * 16 + s

        def load_slabs(lo):
            cp1 = pltpu.async_copy(src_hbm.at[w, pl.ds(lo, PH)], srcs, isem)
            cp2 = pltpu.async_copy(dst_hbm.at[w, pl.ds(lo, PH)], dsts, isem)
            cp1.wait()
            cp2.wait()

        def gather(j, buf, sem):
            pltpu.async_copy(feat_hbm.at[srcs.at[j]], buf, sem)

        def gwait(buf, sem):
            pltpu.make_async_copy(feat_hbm.at[srcs.at[0]], buf, sem).wait()

        def scat(j, buf):
            pltpu.sync_copy(buf, acc.at[dsts.at[j]], add=True)

        def ring():
            @pl.loop(0, PH)
            def _(j):
                pltpu.async_copy(feat_hbm.at[srcs.at[j]], r0, gsem0).wait()
                scat(j, r0)

        cpz = pltpu.async_copy(zeros_hbm, acc.at[pl.ds(s * RPT, RPT)], isem)
        load_slabs(0)
        cpz.wait()
        plsc.subcore_barrier()

        ring()
        load_slabs(PH)
        ring()

        plsc.subcore_barrier()

        @pl.when(c == 0)
        def _():
            pltpu.sync_copy(acc.at[pl.ds(s * RPT, RPT)], o0.at[pl.ds(s * RPT, RPT)])

        @pl.when(c == 1)
        def _():
            pltpu.sync_copy(acc.at[pl.ds(s * RPT, RPT)], o1.at[pl.ds(s * RPT, RPT)])

    return prop


_prop128 = _make_prop(128)


@functools.partial(
    pl.kernel,
    mesh=_MESH,
    out_type=(jax.ShapeDtypeStruct((DEG_PAD,), jnp.float32),) * 2,
    scratch_types=[
        pltpu.VMEM((WB, EB), jnp.int32),
        pltpu.VMEM((EB,), jnp.float32),
        pltpu.VMEM_SHARED((DEG_PAD,), jnp.float32),
        pltpu.SemaphoreType.DMA,
    ],
)
def _deg_kernel(dst_hbm, zeros_hbm, ones_hbm, d0, d1, dsts, onesv, deg, isem):
    c = lax.axis_index("c")
    s = lax.axis_index("s")
    w = c * 16 + s

    cp = pltpu.async_copy(dst_hbm.at[w], dsts, isem)
    pltpu.sync_copy(zeros_hbm, deg.at[pl.ds(s * DEG_RPT, DEG_RPT)])
    pltpu.sync_copy(ones_hbm, onesv)
    cp.wait()
    plsc.subcore_barrier()

    @pl.loop(0, WB)
    def _(j):
        pltpu.sync_copy(onesv, deg.at[dsts.at[j]], add=True)

    plsc.subcore_barrier()

    @pl.when(c == 0)
    def _():
        pltpu.sync_copy(deg.at[pl.ds(s * DEG_RPT, DEG_RPT)], d0.at[pl.ds(s * DEG_RPT, DEG_RPT)])

    @pl.when(c == 1)
    def _():
        pltpu.sync_copy(deg.at[pl.ds(s * DEG_RPT, DEG_RPT)], d1.at[pl.ds(s * DEG_RPT, DEG_RPT)])


BR = 1264


def _tc1_body(d0_ref, d1_ref, x_ref, dis_ref, dis64_ref, xs_ref):
    deg = 1.0 + d0_ref[...] + d1_ref[...]
    dis = lax.rsqrt(deg)
    dis_b = jnp.broadcast_to(dis, (BR, 128))
    dis_ref[...] = dis_b
    dis64_ref[...] = dis_b[:, :64]
    xs_ref[...] = dis_b * x_ref[...]


def _tc1(d0, d1, x):
    return pl.pallas_call(
        _tc1_body,
        grid=(NP // BR,),
        in_specs=[
            pl.BlockSpec((BR, 1), lambda i: (i, 0)),
            pl.BlockSpec((BR, 1), lambda i: (i, 0)),
            pl.BlockSpec((BR, 128), lambda i: (i, 0)),
        ],
        out_specs=[
            pl.BlockSpec((BR, 128), lambda i: (i, 0)),
            pl.BlockSpec((BR, 64), lambda i: (i, 0)),
            pl.BlockSpec((BR, 128), lambda i: (i, 0)),
        ],
        out_shape=[
            jax.ShapeDtypeStruct((NP, 128), jnp.float32),
            jax.ShapeDtypeStruct((NP, 64), jnp.float32),
            jax.ShapeDtypeStruct((NP, 128), jnp.float32),
        ],
    )(d0, d1, x)


def _tc2_body(dis_ref, p0_ref, p1_ref, xs_ref, w1_ref, b1_ref, w2_ref, out_ref):
    s1 = dis_ref[...] * (p0_ref[...] + p1_ref[...] + xs_ref[...])
    h1 = jnp.dot(s1, w1_ref[...], preferred_element_type=jnp.float32) + b1_ref[...]
    h1 = jnp.maximum(h1, 0.0)
    h2 = jnp.dot(h1, w2_ref[...], preferred_element_type=jnp.float32)
    h2s = dis_ref[:, :64] * h2
    out_ref[...] = jnp.concatenate([h2s, jnp.zeros((BR, 64), jnp.float32)], axis=1)


def _tc2(dis_b, p0, p1, xs, W1, b1, W2):
    return pl.pallas_call(
        _tc2_body,
        grid=(NP // BR,),
        in_specs=[
            pl.BlockSpec((BR, 128), lambda i: (i, 0)),
            pl.BlockSpec((BR, 128), lambda i: (i, 0)),
            pl.BlockSpec((BR, 128), lambda i: (i, 0)),
            pl.BlockSpec((BR, 128), lambda i: (i, 0)),
            pl.BlockSpec((128, 256), lambda i: (0, 0)),
            pl.BlockSpec((1, 256), lambda i: (0, 0)),
            pl.BlockSpec((256, 64), lambda i: (0, 0)),
        ],
        out_specs=pl.BlockSpec((BR, 128), lambda i: (i, 0)),
        out_shape=jax.ShapeDtypeStruct((NP, 128), jnp.float32),
    )(dis_b, p0, p1, xs, W1, b1, W2)


def _tc3_body(dis_ref, q0_ref, q1_ref, h2s_ref, b2_ref, out_ref):
    t = q0_ref[...] + q1_ref[...] + h2s_ref[...]
    o = dis_ref[...] * t[:, :64] + b2_ref[...]
    m = jnp.max(o, axis=1, keepdims=True)
    e = jnp.exp(o - m)
    lse = jnp.log(jnp.sum(e, axis=1, keepdims=True))
    out_ref[...] = o - m - lse


def _tc3(dis_b, q0, q1, h2s, b2):
    return pl.pallas_call(
        _tc3_body,
        grid=(NP // BR,),
        in_specs=[
            pl.BlockSpec((BR, 64), lambda i: (i, 0)),
            pl.BlockSpec((BR, 128), lambda i: (i, 0)),
            pl.BlockSpec((BR, 128), lambda i: (i, 0)),
            pl.BlockSpec((BR, 128), lambda i: (i, 0)),
            pl.BlockSpec((1, 64), lambda i: (0, 0)),
        ],
        out_specs=pl.BlockSpec((BR, 64), lambda i: (i, 0)),
        out_shape=jax.ShapeDtypeStruct((NP, 64), jnp.float32),
    )(dis_b, q0, q1, h2s, b2)


def kernel(x, edge_index, W1, b1, W2, b2):
    ei = edge_index.astype(jnp.int32)
    pad_src = jnp.zeros((E_PAD - E,), jnp.int32)
    pad_dst = N + jnp.arange(E_PAD - E, dtype=jnp.int32) % (NP - N)
    src3d = jnp.concatenate([ei[0], pad_src]).reshape(NW, WB, EB)
    dst3d = jnp.concatenate([ei[1], pad_dst]).reshape(NW, WB, EB)

    zeros_deg = jnp.zeros((DEG_RPT,), jnp.float32)
    ones_e = jnp.ones((EB,), jnp.float32)
    zeros128 = jnp.zeros((RPT, 128), jnp.float32)

    xp = jnp.pad(x, ((0, NP - N), (0, 0)))
    d0, d1 = _deg_kernel(dst3d, zeros_deg, ones_e)
    dis_b, dis64, xs = _tc1(d0[:NP, None], d1[:NP, None], xp)

    p0, p1 = _prop128(src3d, dst3d, xs, zeros128)
    h2s = _tc2(dis_b, p0, p1, xs, W1, b1[None, :], W2)

    q0, q1 = _prop128(src3d, dst3d, h2s, zeros128)
    return _tc3(dis64, q0, q1, h2s, b2[None, :])[:N]


# speedup vs baseline: 1.1062x; 1.1062x over previous
import functools

import jax
import jax.numpy as jnp
from jax import lax
from jax.experimental import pallas as pl
from jax.experimental.pallas import tpu as pltpu
from jax.experimental.pallas import tpu_sc as plsc

N = 10000
NP = 10112
E = 320000
EB = 128
NW = 32
WB = 80
PH = 40
E_PAD = NW * WB * EB
RPT = NP // 16
DEG_PAD = 10240
DEG_RPT = DEG_PAD // 16

_MESH = plsc.VectorSubcoreMesh(
    core_axis_name="c", subcore_axis_name="s", num_cores=2, num_subcores=16
)


def _make_prop(feat):
    @functools.partial(
        pl.kernel,
        mesh=_MESH,
        out_type=(jax.ShapeDtypeStruct((NP, feat), jnp.float32),) * 2,
        scratch_types=[
            pltpu.VMEM((PH, EB), jnp.int32),
            pltpu.VMEM((PH, EB), jnp.int32),
            pltpu.VMEM((EB, feat), jnp.float32),
            pltpu.VMEM((EB, feat), jnp.float32),
            pltpu.VMEM_SHARED((NP, feat), jnp.float32),
            pltpu.SemaphoreType.DMA,
            pltpu.SemaphoreType.DMA,
            pltpu.SemaphoreType.DMA,
        ],
    )
    def prop(src_hbm, dst_hbm, feat_hbm, zeros_hbm, o0, o1,
             srcs, dsts, r0, r1, acc, isem, gsem0, gsem1):
        c = lax.axis_index("c")
        s = lax.axis_index("s")
        w = c * 16 + s

        def load_slabs(lo):
            cp1 = pltpu.async_copy(src_hbm.at[w, pl.ds(lo, PH)], srcs, isem)
            cp2 = pltpu.async_copy(dst_hbm.at[w, pl.ds(lo, PH)], dsts, isem)
            cp1.wait()
            cp2.wait()

        def gather(j, buf, sem):
            pltpu.async_copy(feat_hbm.at[srcs.at[j]], buf, sem)

        def gwait(buf, sem):
            pltpu.make_async_copy(feat_hbm.at[srcs.at[0]], buf, sem).wait()

        def scat(j, buf):
            pltpu.sync_copy(buf, acc.at[dsts.at[j]], add=True)

        def ring():
            @pl.loop(0, PH)
            def _(j):
                pltpu.async_copy(feat_hbm.at[srcs.at[j]], r0, gsem0).wait()
                scat(j, r0)

        cpz = pltpu.async_copy(zeros_hbm, acc.at[pl.ds(s * RPT, RPT)], isem)
        load_slabs(0)
        cpz.wait()
        plsc.subcore_barrier()

        ring()
        load_slabs(PH)
        ring()

        plsc.subcore_barrier()

        @pl.when(c == 0)
        def _():
            pltpu.sync_copy(acc.at[pl.ds(s * RPT, RPT)], o0.at[pl.ds(s * RPT, RPT)])

        @pl.when(c == 1)
        def _():
            pltpu.sync_copy(acc.at[pl.ds(s * RPT, RPT)], o1.at[pl.ds(s * RPT, RPT)])

    return prop


_prop128 = _make_prop(128)


@functools.partial(
    pl.kernel,
    mesh=_MESH,
    out_type=(jax.ShapeDtypeStruct((DEG_PAD,), jnp.float32),) * 2,
    scratch_types=[
        pltpu.VMEM((WB, EB), jnp.int32),
        pltpu.VMEM((EB,), jnp.float32),
        pltpu.VMEM_SHARED((DEG_PAD,), jnp.float32),
        pltpu.SemaphoreType.DMA,
    ],
)
def _deg_kernel(dst_hbm, zeros_hbm, ones_hbm, d0, d1, dsts, onesv, deg, isem):
    c = lax.axis_index("c")
    s = lax.axis_index("s")
    w = c * 16 + s

    cp = pltpu.async_copy(dst_hbm.at[w], dsts, isem)
    pltpu.sync_copy(zeros_hbm, deg.at[pl.ds(s * DEG_RPT, DEG_RPT)])
    pltpu.sync_copy(ones_hbm, onesv)
    cp.wait()
    plsc.subcore_barrier()

    @pl.loop(0, WB)
    def _(j):
        pltpu.sync_copy(onesv, deg.at[dsts.at[j]], add=True)

    plsc.subcore_barrier()

    @pl.when(c == 0)
    def _():
        pltpu.sync_copy(deg.at[pl.ds(s * DEG_RPT, DEG_RPT)], d0.at[pl.ds(s * DEG_RPT, DEG_RPT)])

    @pl.when(c == 1)
    def _():
        pltpu.sync_copy(deg.at[pl.ds(s * DEG_RPT, DEG_RPT)], d1.at[pl.ds(s * DEG_RPT, DEG_RPT)])


BR = 1264


def _tc1_body(d0_ref, d1_ref, x_ref, dis_ref, dis64_ref, xs_ref):
    deg = 1.0 + d0_ref[...] + d1_ref[...]
    dis = lax.rsqrt(deg)
    dis_b = jnp.broadcast_to(dis, (BR, 128))
    dis_ref[...] = dis_b
    dis64_ref[...] = dis_b[:, :64]
    xs_ref[...] = dis_b * x_ref[...]


def _tc1(d0, d1, x):
    return pl.pallas_call(
        _tc1_body,
        grid=(NP // BR,),
        in_specs=[
            pl.BlockSpec((BR, 1), lambda i: (i, 0)),
            pl.BlockSpec((BR, 1), lambda i: (i, 0)),
            pl.BlockSpec((BR, 128), lambda i: (i, 0)),
        ],
        out_specs=[
            pl.BlockSpec((BR, 128), lambda i: (i, 0)),
            pl.BlockSpec((BR, 64), lambda i: (i, 0)),
            pl.BlockSpec((BR, 128), lambda i: (i, 0)),
        ],
        out_shape=[
            jax.ShapeDtypeStruct((NP, 128), jnp.float32),
            jax.ShapeDtypeStruct((NP, 64), jnp.float32),
            jax.ShapeDtypeStruct((NP, 128), jnp.float32),
        ],
    )(d0, d1, x)


def _tc2_body(dis_ref, p0_ref, p1_ref, xs_ref, w1_ref, b1_ref, w2_ref, out_ref):
    s1 = dis_ref[...] * (p0_ref[...] + p1_ref[...] + xs_ref[...])
    h1 = jnp.dot(s1, w1_ref[...], preferred_element_type=jnp.float32) + b1_ref[...]
    h1 = jnp.maximum(h1, 0.0)
    h2 = jnp.dot(h1, w2_ref[...], preferred_element_type=jnp.float32)
    h2s = dis_ref[:, :64] * h2
    out_ref[...] = jnp.concatenate([h2s, jnp.zeros((BR, 64), jnp.float32)], axis=1)


def _tc2(dis_b, p0, p1, xs, W1, b1, W2):
    return pl.pallas_call(
        _tc2_body,
        grid=(NP // BR,),
        in_specs=[
            pl.BlockSpec((BR, 128), lambda i: (i, 0)),
            pl.BlockSpec((BR, 128), lambda i: (i, 0)),
            pl.BlockSpec((BR, 128), lambda i: (i, 0)),
            pl.BlockSpec((BR, 128), lambda i: (i, 0)),
            pl.BlockSpec((128, 256), lambda i: (0, 0)),
            pl.BlockSpec((1, 256), lambda i: (0, 0)),
            pl.BlockSpec((256, 64), lambda i: (0, 0)),
        ],
        out_specs=pl.BlockSpec((BR, 128), lambda i: (i, 0)),
        out_shape=jax.ShapeDtypeStruct((NP, 128), jnp.float32),
    )(dis_b, p0, p1, xs, W1, b1, W2)


def _tc3_body(dis_ref, q0_ref, q1_ref, h2s_ref, b2_ref, out_ref):
    t = q0_ref[...] + q1_ref[...] + h2s_ref[...]
    o = dis_ref[...] * t[:, :64] + b2_ref[...]
    m = jnp.max(o, axis=1, keepdims=True)
    e = jnp.exp(o - m)
    lse = jnp.log(jnp.sum(e, axis=1, keepdims=True))
    out_ref[...] = o - m - lse


def _tc3(dis_b, q0, q1, h2s, b2):
    return pl.pallas_call(
        _tc3_body,
        grid=(NP // BR,),
        in_specs=[
            pl.BlockSpec((BR, 64), lambda i: (i, 0)),
            pl.BlockSpec((BR, 128), lambda i: (i, 0)),
            pl.BlockSpec((BR, 128), lambda i: (i, 0)),
            pl.BlockSpec((BR, 128), lambda i: (i, 0)),
            pl.BlockSpec((1, 64), lambda i: (0, 0)),
        ],
        out_specs=pl.BlockSpec((BR, 64), lambda i: (i, 0)),
        out_shape=jax.ShapeDtypeStruct((NP, 64), jnp.float32),
    )(dis_b, q0, q1, h2s, b2)


def kernel(x, edge_index, W1, b1, W2, b2):
    ei = edge_index.astype(jnp.int32)
    ppw = (E_PAD - E) // NW
    pad_src = jnp.zeros((NW, ppw), jnp.int32)
    pad_dst = N + jnp.broadcast_to(
        jnp.arange(ppw, dtype=jnp.int32) % (NP - N), (NW, ppw))
    src3d = jnp.concatenate(
        [ei[0].reshape(NW, E // NW), pad_src], axis=1).reshape(NW, WB, EB)
    dst3d = jnp.concatenate(
        [ei[1].reshape(NW, E // NW), pad_dst], axis=1).reshape(NW, WB, EB)

    zeros_deg = jnp.zeros((DEG_RPT,), jnp.float32)
    ones_e = jnp.ones((EB,), jnp.float32)
    zeros128 = jnp.zeros((RPT, 128), jnp.float32)

    xp = jnp.pad(x, ((0, NP - N), (0, 0)))
    d0, d1 = _deg_kernel(dst3d, zeros_deg, ones_e)
    dis_b, dis64, xs = _tc1(d0[:NP, None], d1[:NP, None], xp)

    p0, p1 = _prop128(src3d, dst3d, xs, zeros128)
    h2s = _tc2(dis_b, p0, p1, xs, W1, b1[None, :], W2)

    q0, q1 = _prop128(src3d, dst3d, h2s, zeros128)
    return _tc3(dis64, q0, q1, h2s, b2[None, :])[:N]


# speedup vs baseline: 1.2251x; 1.1075x over previous
import functools

import jax
import jax.numpy as jnp
from jax import lax
from jax.experimental import pallas as pl
from jax.experimental.pallas import tpu as pltpu
from jax.experimental.pallas import tpu_sc as plsc

N = 10000
NP = 10112
E = 320000
EB = 128
NW = 32
WB = 80
NBLK = NW * WB
E_PAD = NW * WB * EB
RPT = NP // 16
DEG_PAD = 10240
DEG_RPT = DEG_PAD // 16

_MESH = plsc.VectorSubcoreMesh(
    core_axis_name="c", subcore_axis_name="s", num_cores=2, num_subcores=16
)


def _make_prop(feat):
    @functools.partial(
        pl.kernel,
        mesh=_MESH,
        out_type=(jax.ShapeDtypeStruct((NP, feat), jnp.float32),) * 2,
        scratch_types=[
            pltpu.VMEM((EB,), jnp.int32),
            pltpu.VMEM((EB,), jnp.int32),
            pltpu.VMEM((1, EB), jnp.int32),
            pltpu.VMEM((1, EB), jnp.int32),
            pltpu.VMEM((EB, feat), jnp.float32),
            pltpu.VMEM((EB, feat), jnp.float32),
            pltpu.VMEM_SHARED((NP, feat), jnp.float32),
            pltpu.SemaphoreType.DMA,
            pltpu.SemaphoreType.DMA,
            pltpu.SemaphoreType.DMA,
            pltpu.SemaphoreType.DMA,
        ],
    )
    def prop(src_hbm, dst_hbm, feat_hbm, zeros_hbm, o0, o1,
             sv0, sv1, dv0, dv1, r0, r1, acc,
             isem0, isem1, gsem0, gsem1):
        c = lax.axis_index("c")
        s = lax.axis_index("s")
        w = c * 16 + s

        sv = (sv0, sv1)
        dv = (dv0, dv1)

        def idx_load(j, k, sem):
            b = w + j * NW
            pltpu.async_copy(src_hbm.at[b], sv[k], sem)
            pltpu.async_copy(dst_hbm.at[b], dv[k].at[0], sem)

        def idx_wait(k, sem):
            pltpu.make_async_copy(src_hbm.at[0], sv[k], sem).wait()
            pltpu.make_async_copy(src_hbm.at[0], dv[k].at[0], sem).wait()

        def gather(k, buf, sem):
            pltpu.async_copy(feat_hbm.at[sv[k]], buf, sem)

        def gwait(buf, sem):
            pltpu.make_async_copy(feat_hbm.at[sv0], buf, sem).wait()

        def scat(k, buf):
            pltpu.sync_copy(buf, acc.at[dv[k].at[0]], add=True)

        idx_load(0, 0, isem0)
        cpz = pltpu.async_copy(zeros_hbm, acc.at[pl.ds(s * RPT, RPT)], isem1)
        cpz.wait()
        plsc.subcore_barrier()

        idx_wait(0, isem0)
        gather(0, r0, gsem0)
        idx_load(1, 1, isem1)

        @pl.loop(0, WB // 2)
        def _(i):
            j = 2 * i
            idx_wait(1, isem1)
            gather(1, r1, gsem1)
            gwait(r0, gsem0)
            scat(0, r0)
            @pl.when(j + 2 < WB)
            def _():
                idx_load(j + 2, 0, isem0)
                idx_wait(0, isem0)
                gather(0, r0, gsem0)
            gwait(r1, gsem1)
            scat(1, r1)
            @pl.when(j + 3 < WB)
            def _():
                idx_load(j + 3, 1, isem1)

        plsc.subcore_barrier()

        @pl.when(c == 0)
        def _():
            pltpu.sync_copy(acc.at[pl.ds(s * RPT, RPT)], o0.at[pl.ds(s * RPT, RPT)])

        @pl.when(c == 1)
        def _():
            pltpu.sync_copy(acc.at[pl.ds(s * RPT, RPT)], o1.at[pl.ds(s * RPT, RPT)])

    return prop


_prop128 = _make_prop(128)


@functools.partial(
    pl.kernel,
    mesh=_MESH,
    out_type=(jax.ShapeDtypeStruct((DEG_PAD,), jnp.float32),) * 2,
    scratch_types=[
        pltpu.VMEM((WB, EB), jnp.int32),
        pltpu.VMEM((EB,), jnp.float32),
        pltpu.VMEM_SHARED((DEG_PAD,), jnp.float32),
        pltpu.SemaphoreType.DMA,
    ],
)
def _deg_kernel(dst_hbm, zeros_hbm, ones_hbm, d0, d1, dsts, onesv, deg, isem):
    c = lax.axis_index("c")
    s = lax.axis_index("s")
    w = c * 16 + s

    cp = pltpu.async_copy(dst_hbm.at[w], dsts, isem)
    pltpu.sync_copy(zeros_hbm, deg.at[pl.ds(s * DEG_RPT, DEG_RPT)])
    pltpu.sync_copy(ones_hbm, onesv)
    cp.wait()
    plsc.subcore_barrier()

    @pl.loop(0, WB)
    def _(j):
        pltpu.sync_copy(onesv, deg.at[dsts.at[j]], add=True)

    plsc.subcore_barrier()

    @pl.when(c == 0)
    def _():
        pltpu.sync_copy(deg.at[pl.ds(s * DEG_RPT, DEG_RPT)], d0.at[pl.ds(s * DEG_RPT, DEG_RPT)])

    @pl.when(c == 1)
    def _():
        pltpu.sync_copy(deg.at[pl.ds(s * DEG_RPT, DEG_RPT)], d1.at[pl.ds(s * DEG_RPT, DEG_RPT)])


BR = 1264


def _tc1_body(d0_ref, d1_ref, x_ref, dis_ref, dis64_ref, xs_ref):
    deg = 1.0 + d0_ref[...] + d1_ref[...]
    dis = lax.rsqrt(deg)
    dis_b = jnp.broadcast_to(dis, (BR, 128))
    dis_ref[...] = dis_b
    dis64_ref[...] = dis_b[:, :64]
    xs_ref[...] = dis_b * x_ref[...]


def _tc1(d0, d1, x):
    return pl.pallas_call(
        _tc1_body,
        grid=(NP // BR,),
        in_specs=[
            pl.BlockSpec((BR, 1), lambda i: (i, 0)),
            pl.BlockSpec((BR, 1), lambda i: (i, 0)),
            pl.BlockSpec((BR, 128), lambda i: (i, 0)),
        ],
        out_specs=[
            pl.BlockSpec((BR, 128), lambda i: (i, 0)),
            pl.BlockSpec((BR, 64), lambda i: (i, 0)),
            pl.BlockSpec((BR, 128), lambda i: (i, 0)),
        ],
        out_shape=[
            jax.ShapeDtypeStruct((NP, 128), jnp.float32),
            jax.ShapeDtypeStruct((NP, 64), jnp.float32),
            jax.ShapeDtypeStruct((NP, 128), jnp.float32),
        ],
    )(d0, d1, x)


def _tc2_body(dis_ref, p0_ref, p1_ref, xs_ref, w1_ref, b1_ref, w2_ref, out_ref):
    s1 = dis_ref[...] * (p0_ref[...] + p1_ref[...] + xs_ref[...])
    h1 = jnp.dot(s1, w1_ref[...], preferred_element_type=jnp.float32) + b1_ref[...]
    h1 = jnp.maximum(h1, 0.0)
    h2 = jnp.dot(h1, w2_ref[...], preferred_element_type=jnp.float32)
    h2s = dis_ref[:, :64] * h2
    out_ref[...] = jnp.concatenate([h2s, jnp.zeros((BR, 64), jnp.float32)], axis=1)


def _tc2(dis_b, p0, p1, xs, W1, b1, W2):
    return pl.pallas_call(
        _tc2_body,
        grid=(NP // BR,),
        in_specs=[
            pl.BlockSpec((BR, 128), lambda i: (i, 0)),
            pl.BlockSpec((BR, 128), lambda i: (i, 0)),
            pl.BlockSpec((BR, 128), lambda i: (i, 0)),
            pl.BlockSpec((BR, 128), lambda i: (i, 0)),
            pl.BlockSpec((128, 256), lambda i: (0, 0)),
            pl.BlockSpec((1, 256), lambda i: (0, 0)),
            pl.BlockSpec((256, 64), lambda i: (0, 0)),
        ],
        out_specs=pl.BlockSpec((BR, 128), lambda i: (i, 0)),
        out_shape=jax.ShapeDtypeStruct((NP, 128), jnp.float32),
    )(dis_b, p0, p1, xs, W1, b1, W2)


def _tc3_body(dis_ref, q0_ref, q1_ref, h2s_ref, b2_ref, out_ref):
    t = q0_ref[...] + q1_ref[...] + h2s_ref[...]
    o = dis_ref[...] * t[:, :64] + b2_ref[...]
    m = jnp.max(o, axis=1, keepdims=True)
    e = jnp.exp(o - m)
    lse = jnp.log(jnp.sum(e, axis=1, keepdims=True))
    out_ref[...] = o - m - lse


def _tc3(dis_b, q0, q1, h2s, b2):
    return pl.pallas_call(
        _tc3_body,
        grid=(NP // BR,),
        in_specs=[
            pl.BlockSpec((BR, 64), lambda i: (i, 0)),
            pl.BlockSpec((BR, 128), lambda i: (i, 0)),
            pl.BlockSpec((BR, 128), lambda i: (i, 0)),
            pl.BlockSpec((BR, 128), lambda i: (i, 0)),
            pl.BlockSpec((1, 64), lambda i: (0, 0)),
        ],
        out_specs=pl.BlockSpec((BR, 64), lambda i: (i, 0)),
        out_shape=jax.ShapeDtypeStruct((NP, 64), jnp.float32),
    )(dis_b, q0, q1, h2s, b2)


def kernel(x, edge_index, W1, b1, W2, b2):
    ei = edge_index.astype(jnp.int32)
    npad = E_PAD - E
    pad_src = jnp.zeros((npad,), jnp.int32)
    pad_dst = N + jnp.arange(npad, dtype=jnp.int32) % (NP - N)
    src_flat = jnp.concatenate([ei[0], pad_src])
    dst_flat = jnp.concatenate([ei[1], pad_dst])
    src2d = src_flat.reshape(NBLK, EB)
    dst2d = dst_flat.reshape(NBLK, EB)
    dst3d = dst_flat.reshape(NW, WB, EB)

    zeros_deg = jnp.zeros((DEG_RPT,), jnp.float32)
    ones_e = jnp.ones((EB,), jnp.float32)
    zeros128 = jnp.zeros((RPT, 128), jnp.float32)

    xp = jnp.pad(x, ((0, NP - N), (0, 0)))
    d0, d1 = _deg_kernel(dst3d, zeros_deg, ones_e)
    dis_b, dis64, xs = _tc1(d0[:NP, None], d1[:NP, None], xp)

    p0, p1 = _prop128(src2d, dst2d, xs, zeros128)
    h2s = _tc2(dis_b, p0, p1, xs, W1, b1[None, :], W2)

    q0, q1 = _prop128(src2d, dst2d, h2s, zeros128)
    return _tc3(dis64, q0, q1, h2s, b2[None, :])[:N]


# speedup vs baseline: 3.2208x; 2.6289x over previous
import functools

import jax
import jax.numpy as jnp
from jax import lax
from jax.experimental import pallas as pl
from jax.experimental.pallas import tpu as pltpu
from jax.experimental.pallas import tpu_sc as plsc

N = 10000
NP = 10112
E = 320000
EB = 128
NW = 32
WB = 80
NBLK = NW * WB
E_PAD = NW * WB * EB
RPT = NP // 16
DEG_PAD = 10240
DEG_RPT = DEG_PAD // 16

_MESH = plsc.VectorSubcoreMesh(
    core_axis_name="c", subcore_axis_name="s", num_cores=2, num_subcores=16
)


def _make_prop(feat):
    @functools.partial(
        pl.kernel,
        mesh=_MESH,
        out_type=(jax.ShapeDtypeStruct((NP, feat), jnp.float32),) * 2,
        scratch_types=[
            pltpu.VMEM((EB,), jnp.int32),
            pltpu.VMEM((EB,), jnp.int32),
            pltpu.VMEM((1, EB), jnp.int32),
            pltpu.VMEM((1, EB), jnp.int32),
            pltpu.VMEM((EB, feat), jnp.float32),
            pltpu.VMEM((EB, feat), jnp.float32),
            pltpu.VMEM_SHARED((NP, feat), jnp.float32),
            pltpu.SemaphoreType.DMA,
            pltpu.SemaphoreType.DMA,
            pltpu.SemaphoreType.DMA,
            pltpu.SemaphoreType.DMA,
        ],
    )
    def prop(src_hbm, dst_hbm, feat_hbm, zeros_hbm, o0, o1,
             sv0, sv1, dv0, dv1, r0, r1, acc,
             isem0, isem1, gsem0, gsem1):
        c = lax.axis_index("c")
        s = lax.axis_index("s")
        w = c * 16 + s

        sv = (sv0, sv1)
        dv = (dv0, dv1)

        def idx_load(j, k, sem):
            b = w + j * NW
            pltpu.async_copy(src_hbm.at[b], sv[k], sem)
            pltpu.async_copy(dst_hbm.at[b], dv[k].at[0], sem)

        def idx_wait(k, sem):
            pltpu.make_async_copy(src_hbm.at[0], sv[k], sem).wait()
            pltpu.make_async_copy(src_hbm.at[0], dv[k].at[0], sem).wait()

        def gather(k, buf, sem):
            pltpu.async_copy(feat_hbm.at[sv[k]], buf, sem)

        def gwait(buf, sem):
            pltpu.make_async_copy(feat_hbm.at[sv0], buf, sem).wait()

        def scat(k, buf):
            pltpu.sync_copy(buf, acc.at[dv[k].at[0]], add=True)

        idx_load(0, 0, isem0)
        cpz = pltpu.async_copy(zeros_hbm, acc.at[pl.ds(s * RPT, RPT)], isem1)
        cpz.wait()
        plsc.subcore_barrier()

        idx_wait(0, isem0)
        gather(0, r0, gsem0)
        idx_load(1, 1, isem1)

        @pl.loop(0, WB // 2)
        def _(i):
            j = 2 * i
            idx_wait(1, isem1)
            gather(1, r1, gsem1)
            gwait(r0, gsem0)
            scat(0, r0)
            @pl.when(j + 2 < WB)
            def _():
                idx_load(j + 2, 0, isem0)
                idx_wait(0, isem0)
                gather(0, r0, gsem0)
            gwait(r1, gsem1)
            scat(1, r1)
            @pl.when(j + 3 < WB)
            def _():
                idx_load(j + 3, 1, isem1)

        plsc.subcore_barrier()

        @pl.when(c == 0)
        def _():
            pltpu.sync_copy(acc.at[pl.ds(s * RPT, RPT)], o0.at[pl.ds(s * RPT, RPT)])

        @pl.when(c == 1)
        def _():
            pltpu.sync_copy(acc.at[pl.ds(s * RPT, RPT)], o1.at[pl.ds(s * RPT, RPT)])

    return prop


_prop128 = _make_prop(128)


@functools.partial(
    pl.kernel,
    mesh=_MESH,
    out_type=(jax.ShapeDtypeStruct((DEG_PAD,), jnp.float32),) * 2,
    scratch_types=[
        pltpu.VMEM((WB, EB), jnp.int32),
        pltpu.VMEM((EB,), jnp.float32),
        pltpu.VMEM_SHARED((DEG_PAD,), jnp.float32),
        pltpu.SemaphoreType.DMA,
    ],
)
def _deg_kernel(dst_hbm, zeros_hbm, ones_hbm, d0, d1, dsts, onesv, deg, isem):
    c = lax.axis_index("c")
    s = lax.axis_index("s")
    w = c * 16 + s

    cp = pltpu.async_copy(dst_hbm.at[w], dsts, isem)
    pltpu.sync_copy(zeros_hbm, deg.at[pl.ds(s * DEG_RPT, DEG_RPT)])
    pltpu.sync_copy(ones_hbm, onesv)
    cp.wait()
    plsc.subcore_barrier()

    @pl.loop(0, WB)
    def _(j):
        pltpu.sync_copy(onesv, deg.at[dsts.at[j]], add=True)

    plsc.subcore_barrier()

    @pl.when(c == 0)
    def _():
        pltpu.sync_copy(deg.at[pl.ds(s * DEG_RPT, DEG_RPT)], d0.at[pl.ds(s * DEG_RPT, DEG_RPT)])

    @pl.when(c == 1)
    def _():
        pltpu.sync_copy(deg.at[pl.ds(s * DEG_RPT, DEG_RPT)], d1.at[pl.ds(s * DEG_RPT, DEG_RPT)])


BR = 1264


def _tc1_body(d0_ref, d1_ref, x_ref, dis_ref, dis64_ref, xs_ref):
    deg = 1.0 + d0_ref[...] + d1_ref[...]
    dis = lax.rsqrt(deg)
    dis_b = jnp.broadcast_to(dis, (BR, 128))
    dis_ref[...] = dis_b
    dis64_ref[...] = dis_b[:, :64]
    xs_ref[...] = dis_b * x_ref[...]


def _tc1(d0, d1, x):
    return pl.pallas_call(
        _tc1_body,
        grid=(NP // BR,),
        in_specs=[
            pl.BlockSpec((BR, 1), lambda i: (i, 0)),
            pl.BlockSpec((BR, 1), lambda i: (i, 0)),
            pl.BlockSpec((BR, 128), lambda i: (i, 0)),
        ],
        out_specs=[
            pl.BlockSpec((BR, 128), lambda i: (i, 0)),
            pl.BlockSpec((BR, 64), lambda i: (i, 0)),
            pl.BlockSpec((BR, 128), lambda i: (i, 0)),
        ],
        out_shape=[
            jax.ShapeDtypeStruct((NP, 128), jnp.float32),
            jax.ShapeDtypeStruct((NP, 64), jnp.float32),
            jax.ShapeDtypeStruct((NP, 128), jnp.float32),
        ],
    )(d0, d1, x)


def _tc2_body(dis_ref, p0_ref, p1_ref, xs_ref, w1_ref, b1_ref, w2_ref, out_ref):
    s1 = dis_ref[...] * (p0_ref[...] + p1_ref[...] + xs_ref[...])
    h1 = jnp.dot(s1, w1_ref[...], preferred_element_type=jnp.float32) + b1_ref[...]
    h1 = jnp.maximum(h1, 0.0)
    h2 = jnp.dot(h1, w2_ref[...], preferred_element_type=jnp.float32)
    h2s = dis_ref[:, :64] * h2
    out_ref[...] = jnp.concatenate([h2s, jnp.zeros((BR, 64), jnp.float32)], axis=1)


def _tc2(dis_b, p0, p1, xs, W1, b1, W2):
    return pl.pallas_call(
        _tc2_body,
        grid=(NP // BR,),
        in_specs=[
            pl.BlockSpec((BR, 128), lambda i: (i, 0)),
            pl.BlockSpec((BR, 128), lambda i: (i, 0)),
            pl.BlockSpec((BR, 128), lambda i: (i, 0)),
            pl.BlockSpec((BR, 128), lambda i: (i, 0)),
            pl.BlockSpec((128, 256), lambda i: (0, 0)),
            pl.BlockSpec((1, 256), lambda i: (0, 0)),
            pl.BlockSpec((256, 64), lambda i: (0, 0)),
        ],
        out_specs=pl.BlockSpec((BR, 128), lambda i: (i, 0)),
        out_shape=jax.ShapeDtypeStruct((NP, 128), jnp.float32),
    )(dis_b, p0, p1, xs, W1, b1, W2)


def _tc3_body(dis_ref, q0_ref, q1_ref, h2s_ref, b2_ref, out_ref):
    t = q0_ref[...] + q1_ref[...] + h2s_ref[...]
    o = dis_ref[...] * t[:, :64] + b2_ref[...]
    m = jnp.max(o, axis=1, keepdims=True)
    e = jnp.exp(o - m)
    lse = jnp.log(jnp.sum(e, axis=1, keepdims=True))
    out_ref[...] = o - m - lse


def _tc3(dis_b, q0, q1, h2s, b2):
    return pl.pallas_call(
        _tc3_body,
        grid=(NP // BR,),
        in_specs=[
            pl.BlockSpec((BR, 64), lambda i: (i, 0)),
            pl.BlockSpec((BR, 128), lambda i: (i, 0)),
            pl.BlockSpec((BR, 128), lambda i: (i, 0)),
            pl.BlockSpec((BR, 128), lambda i: (i, 0)),
            pl.BlockSpec((1, 64), lambda i: (0, 0)),
        ],
        out_specs=pl.BlockSpec((BR, 64), lambda i: (i, 0)),
        out_shape=jax.ShapeDtypeStruct((NP, 64), jnp.float32),
    )(dis_b, q0, q1, h2s, b2)


def kernel(x, edge_index, W1, b1, W2, b2):
    ei = edge_index.astype(jnp.int32)
    npad = E_PAD - E
    pad_src = jnp.arange(npad, dtype=jnp.int32) % N
    pad_dst = N + jnp.arange(npad, dtype=jnp.int32) % (NP - N)
    src_flat = jnp.concatenate([ei[0], pad_src])
    dst_flat = jnp.concatenate([ei[1], pad_dst])
    src2d = src_flat.reshape(NBLK, EB)
    dst2d = dst_flat.reshape(NBLK, EB)
    dst3d = dst_flat.reshape(NW, WB, EB)

    zeros_deg = jnp.zeros((DEG_RPT,), jnp.float32)
    ones_e = jnp.ones((EB,), jnp.float32)
    zeros128 = jnp.zeros((RPT, 128), jnp.float32)

    xp = jnp.pad(x, ((0, NP - N), (0, 0)))
    d0, d1 = _deg_kernel(dst3d, zeros_deg, ones_e)
    dis_b, dis64, xs = _tc1(d0[:NP, None], d1[:NP, None], xp)

    p0, p1 = _prop128(src2d, dst2d, xs, zeros128)
    h2s = _tc2(dis_b, p0, p1, xs, W1, b1[None, :], W2)

    q0, q1 = _prop128(src2d, dst2d, h2s, zeros128)
    return _tc3(dis64, q0, q1, h2s, b2[None, :])[:N]


# speedup vs baseline: 3.2867x; 1.0205x over previous
import functools

import jax
import jax.numpy as jnp
from jax import lax
from jax.experimental import pallas as pl
from jax.experimental.pallas import tpu as pltpu
from jax.experimental.pallas import tpu_sc as plsc

N = 10000
NP = 10112
E = 320000
EB = 128
NW = 32
WB = 80
NBLK = NW * WB
E_PAD = NW * WB * EB
RPT = NP // 16
DEG_PAD = 10240
DEG_RPT = DEG_PAD // 16

_MESH = plsc.VectorSubcoreMesh(
    core_axis_name="c", subcore_axis_name="s", num_cores=2, num_subcores=16
)


def _make_prop(feat):
    @functools.partial(
        pl.kernel,
        mesh=_MESH,
        out_type=(jax.ShapeDtypeStruct((NP, feat), jnp.float32),) * 2,
        scratch_types=[
            pltpu.VMEM((EB,), jnp.int32),
            pltpu.VMEM((EB,), jnp.int32),
            pltpu.VMEM((1, EB), jnp.int32),
            pltpu.VMEM((1, EB), jnp.int32),
            pltpu.VMEM((EB, feat), jnp.float32),
            pltpu.VMEM((EB, feat), jnp.float32),
            pltpu.VMEM_SHARED((NP, feat), jnp.float32),
            pltpu.SemaphoreType.DMA,
            pltpu.SemaphoreType.DMA,
            pltpu.SemaphoreType.DMA,
            pltpu.SemaphoreType.DMA,
        ],
    )
    def prop(src_hbm, dst_hbm, feat_hbm, zeros_hbm, o0, o1,
             sv0, sv1, dv0, dv1, r0, r1, acc,
             isem0, isem1, gsem0, gsem1):
        c = lax.axis_index("c")
        s = lax.axis_index("s")
        w = c * 16 + s

        sv = (sv0, sv1)
        dv = (dv0, dv1)

        def idx_load(j, k, sem):
            b = w + j * NW
            pltpu.async_copy(src_hbm.at[b], sv[k], sem)
            pltpu.async_copy(dst_hbm.at[b], dv[k].at[0], sem)

        def idx_wait(k, sem):
            pltpu.make_async_copy(src_hbm.at[0], sv[k], sem).wait()
            pltpu.make_async_copy(src_hbm.at[0], dv[k].at[0], sem).wait()

        def gather(k, buf, sem):
            pltpu.async_copy(feat_hbm.at[sv[k]], buf, sem)

        def gwait(buf, sem):
            pltpu.make_async_copy(feat_hbm.at[sv0], buf, sem).wait()

        def scat(k, buf):
            pltpu.sync_copy(buf, acc.at[dv[k].at[0]], add=True)

        idx_load(0, 0, isem0)
        cpz = pltpu.async_copy(zeros_hbm, acc.at[pl.ds(s * RPT, RPT)], isem1)
        cpz.wait()
        plsc.subcore_barrier()

        idx_wait(0, isem0)
        gather(0, r0, gsem0)
        idx_load(1, 1, isem1)

        @pl.loop(0, WB // 2)
        def _(i):
            j = 2 * i
            idx_wait(1, isem1)
            gather(1, r1, gsem1)
            gwait(r0, gsem0)
            scat(0, r0)
            @pl.when(j + 2 < WB)
            def _():
                idx_load(j + 2, 0, isem0)
                idx_wait(0, isem0)
                gather(0, r0, gsem0)
            gwait(r1, gsem1)
            scat(1, r1)
            @pl.when(j + 3 < WB)
            def _():
                idx_load(j + 3, 1, isem1)

        plsc.subcore_barrier()

        @pl.when(c == 0)
        def _():
            pltpu.sync_copy(acc.at[pl.ds(s * RPT, RPT)], o0.at[pl.ds(s * RPT, RPT)])

        @pl.when(c == 1)
        def _():
            pltpu.sync_copy(acc.at[pl.ds(s * RPT, RPT)], o1.at[pl.ds(s * RPT, RPT)])

    return prop


_prop128 = _make_prop(128)


@functools.partial(
    pl.kernel,
    mesh=_MESH,
    out_type=(jax.ShapeDtypeStruct((DEG_PAD,), jnp.float32),) * 2,
    scratch_types=[
        pltpu.VMEM((WB, EB), jnp.int32),
        pltpu.VMEM((EB,), jnp.float32),
        pltpu.VMEM_SHARED((DEG_PAD,), jnp.float32),
        pltpu.SemaphoreType.DMA,
    ],
)
def _deg_kernel(dst_hbm, zeros_hbm, ones_hbm, d0, d1, dsts, onesv, deg, isem):
    c = lax.axis_index("c")
    s = lax.axis_index("s")
    w = c * 16 + s

    cp = pltpu.async_copy(dst_hbm.at[w], dsts, isem)
    pltpu.sync_copy(zeros_hbm, deg.at[pl.ds(s * DEG_RPT, DEG_RPT)])
    pltpu.sync_copy(ones_hbm, onesv)
    cp.wait()
    plsc.subcore_barrier()

    @pl.loop(0, WB)
    def _(j):
        pltpu.sync_copy(onesv, deg.at[dsts.at[j]], add=True)

    plsc.subcore_barrier()

    @pl.when(c == 0)
    def _():
        pltpu.sync_copy(deg.at[pl.ds(s * DEG_RPT, DEG_RPT)], d0.at[pl.ds(s * DEG_RPT, DEG_RPT)])

    @pl.when(c == 1)
    def _():
        pltpu.sync_copy(deg.at[pl.ds(s * DEG_RPT, DEG_RPT)], d1.at[pl.ds(s * DEG_RPT, DEG_RPT)])


BR = 1264


def _tc1_body(d0_ref, d1_ref, x_ref, dis_ref, dis64_ref, xs_ref):
    deg = 1.0 + d0_ref[...] + d1_ref[...]
    dis = lax.rsqrt(deg)
    dis_b = jnp.broadcast_to(dis, (BR, 128))
    dis_ref[...] = dis_b
    dis64_ref[...] = dis_b[:, :64]
    xs_ref[...] = dis_b * x_ref[...]


def _tc1(d0, d1, x):
    return pl.pallas_call(
        _tc1_body,
        grid=(NP // BR,),
        in_specs=[
            pl.BlockSpec((BR, 1), lambda i: (i, 0)),
            pl.BlockSpec((BR, 1), lambda i: (i, 0)),
            pl.BlockSpec((BR, 128), lambda i: (i, 0)),
        ],
        out_specs=[
            pl.BlockSpec((BR, 128), lambda i: (i, 0)),
            pl.BlockSpec((BR, 64), lambda i: (i, 0)),
            pl.BlockSpec((BR, 128), lambda i: (i, 0)),
        ],
        out_shape=[
            jax.ShapeDtypeStruct((NP, 128), jnp.float32),
            jax.ShapeDtypeStruct((NP, 64), jnp.float32),
            jax.ShapeDtypeStruct((NP, 128), jnp.float32),
        ],
    )(d0, d1, x)


def _tc2_body(dis_ref, p0_ref, p1_ref, xs_ref, w1_ref, b1_ref, w2_ref, out_ref):
    s1 = dis_ref[...] * (p0_ref[...] + p1_ref[...] + xs_ref[...])
    h1 = jnp.dot(s1, w1_ref[...], preferred_element_type=jnp.float32) + b1_ref[...]
    h1 = jnp.maximum(h1, 0.0)
    h2 = jnp.dot(h1, w2_ref[...], preferred_element_type=jnp.float32)
    h2s = dis_ref[:, :64] * h2
    out_ref[...] = jnp.concatenate([h2s, jnp.zeros((BR, 64), jnp.float32)], axis=1)


def _tc2(dis_b, p0, p1, xs, W1, b1, W2):
    return pl.pallas_call(
        _tc2_body,
        grid=(NP // BR,),
        in_specs=[
            pl.BlockSpec((BR, 128), lambda i: (i, 0)),
            pl.BlockSpec((BR, 128), lambda i: (i, 0)),
            pl.BlockSpec((BR, 128), lambda i: (i, 0)),
            pl.BlockSpec((BR, 128), lambda i: (i, 0)),
            pl.BlockSpec((128, 256), lambda i: (0, 0)),
            pl.BlockSpec((1, 256), lambda i: (0, 0)),
            pl.BlockSpec((256, 64), lambda i: (0, 0)),
        ],
        out_specs=pl.BlockSpec((BR, 128), lambda i: (i, 0)),
        out_shape=jax.ShapeDtypeStruct((NP, 128), jnp.float32),
    )(dis_b, p0, p1, xs, W1, b1, W2)


def _tc3_body(dis_ref, q0_ref, q1_ref, h2s_ref, b2_ref, out_ref):
    t = q0_ref[...] + q1_ref[...] + h2s_ref[...]
    o = dis_ref[...] * t[:, :64] + b2_ref[...]
    m = jnp.max(o, axis=1, keepdims=True)
    e = jnp.exp(o - m)
    lse = jnp.log(jnp.sum(e, axis=1, keepdims=True))
    out_ref[...] = o - m - lse


def _tc3(dis_b, q0, q1, h2s, b2):
    return pl.pallas_call(
        _tc3_body,
        grid=(NP // BR,),
        in_specs=[
            pl.BlockSpec((BR, 64), lambda i: (i, 0)),
            pl.BlockSpec((BR, 128), lambda i: (i, 0)),
            pl.BlockSpec((BR, 128), lambda i: (i, 0)),
            pl.BlockSpec((BR, 128), lambda i: (i, 0)),
            pl.BlockSpec((1, 64), lambda i: (0, 0)),
        ],
        out_specs=pl.BlockSpec((BR, 64), lambda i: (i, 0)),
        out_shape=jax.ShapeDtypeStruct((N, 64), jnp.float32),
    )(dis_b, q0, q1, h2s, b2)


def kernel(x, edge_index, W1, b1, W2, b2):
    ei = edge_index.astype(jnp.int32)
    npad = E_PAD - E
    pad_src = jnp.arange(npad, dtype=jnp.int32) % N
    pad_dst = N + jnp.arange(npad, dtype=jnp.int32) % (NP - N)
    src_flat = jnp.concatenate([ei[0], pad_src])
    dst_flat = jnp.concatenate([ei[1], pad_dst])
    src2d = src_flat.reshape(NBLK, EB)
    dst2d = dst_flat.reshape(NBLK, EB)
    dst3d = dst_flat.reshape(NW, WB, EB)

    zeros_deg = jnp.zeros((DEG_RPT,), jnp.float32)
    ones_e = jnp.ones((EB,), jnp.float32)
    zeros128 = jnp.zeros((RPT, 128), jnp.float32)

    d0, d1 = _deg_kernel(dst3d, zeros_deg, ones_e)
    dis_b, dis64, xs = _tc1(d0[:, None], d1[:, None], x)

    p0, p1 = _prop128(src2d, dst2d, xs, zeros128)
    h2s = _tc2(dis_b, p0, p1, xs, W1, b1[None, :], W2)

    q0, q1 = _prop128(src2d, dst2d, h2s, zeros128)
    return _tc3(dis64, q0, q1, h2s, b2[None, :])
